# baseline (speedup 1.0000x reference)
import functools

import jax
import jax.numpy as jnp
from jax.experimental import pallas as pl
from jax.experimental.pallas import tpu as pltpu

C = 64
G = 8
CPB = 4
TM = 512

_dot = functools.partial(jax.lax.dot_general,
                         preferred_element_type=jnp.float32,
                         precision=None)


def _mm(a, b):
    return _dot(a, b, (((1,), (0,)), ((), ())))


def _mmT(a, b):
    return _dot(a, b, (((1,), (1,)), ((), ())))


def _mTm(a, b):
    return _dot(a, b, (((0,), (0,)), ((), ())))


def _proj_in_body(x_ref, ww_ref, wb_ref, e_ref, v_ref):
    x = x_ref[...]
    bexp = _mm(jax.nn.sigmoid(_mmT(x, wb_ref[...])), e_ref[...])
    v_ref[...] = _mmT(x, ww_ref[...]) * bexp


def _proj_out_body(o_ref, out_ref, wr_ref, y_ref):
    y_ref[...] = out_ref[...] + _mmT(o_ref[...], wr_ref[...])


def _scan_body(x_ref, v_ref, wb_ref, e_ref, L_ref, dexp_ref, dw_ref, ct_ref,
               o_ref, S_ref, prev_ref, *, H, d):
    n = pl.program_id(1)

    @pl.when(n == 0)
    def _():
        S_ref[...] = jnp.zeros_like(S_ref)
        prev_ref[...] = jnp.zeros_like(prev_ref)

    ii = jax.lax.broadcasted_iota(jnp.int32, (C, C), 0)
    jj = jax.lax.broadcasted_iota(jnp.int32, (C, C), 1)
    strict = ii > jj

    hg = jax.lax.rem(pl.program_id(0), jnp.int32(H // G))

    S_all = S_ref[...]
    prev_all = prev_ref[...]
    x_all = x_ref[...]
    v_all = v_ref[...]
    b_all = _mm(jax.nn.sigmoid(_mmT(x_all, wb_ref[...])), e_ref[...])
    L_all = L_ref[...]
    de_all = dexp_ref[...]
    dw_all = dw_ref[...]

    rk_l, wk_l, Mk_l, X_l, attn_l = {}, {}, {}, {}, {}
    for c in range(CPB):
        rs = slice(c * C, (c + 1) * C)
        for g in range(G):
            sl = slice(g * d, (g + 1) * d)
            xh = x_all[rs, sl]
            n2 = jnp.sum(xh * xh, axis=1, keepdims=True)
            rk_l[c, g] = xh * jax.lax.rsqrt(jnp.maximum(n2, 1e-24))
    for c in range(CPB):
        for g in range(G):
            prev = (prev_all[g:g + 1, :] if c == 0
                    else rk_l[c - 1, g][C - 1:C, :])
            wk_l[c, g] = jnp.concatenate([prev, rk_l[c, g][:C - 1, :]], axis=0)
    for c in range(CPB):
        rs = slice(c * C, (c + 1) * C)
        for g in range(G):
            sl = slice(g * d, (g + 1) * d)
            wkb = wk_l[c, g] * b_all[rs, sl]
            pre = _mmT(jnp.concatenate([wkb, rk_l[c, g]], axis=0), wk_l[c, g])
            Mk_l[c, g] = jnp.where(strict, -pre[:C] * L_all[g], 0.0)
            attn_l[c, g] = pre[C:] * L_all[g]
            X_l[c, g] = jnp.concatenate([v_all[rs, sl], wkb * de_all[g]],
                                        axis=1)
    for c in range(CPB):
        for g in range(G):
            X_l[c, g] = X_l[c, g] + _mm(Mk_l[c, g], X_l[c, g])
    for _ in range(5):
        for c in range(CPB):
            for g in range(G):
                Mk_l[c, g] = _mm(Mk_l[c, g], Mk_l[c, g])
        for c in range(CPB):
            for g in range(G):
                X_l[c, g] = X_l[c, g] + _mm(Mk_l[c, g], X_l[c, g])

    S_vals = [S_all[g * d:(g + 1) * d, :] for g in range(G)]
    for c in range(CPB):
        rs = slice(c * C, (c + 1) * C)
        vn_c = []
        for g in range(G):
            vn_c.append(X_l[c, g][:, :d] - _mm(X_l[c, g][:, d:], S_vals[g]))
        for g in range(G):
            sl = slice(g * d, (g + 1) * d)
            o_ref[rs, sl] = _mm(
                jnp.concatenate([rk_l[c, g] * de_all[g], attn_l[c, g]], axis=1),
                jnp.concatenate([S_vals[g], vn_c[g]], axis=0))
        for g in range(G):
            S_vals[g] = (ct_ref[hg * G + g] * S_vals[g]
                         + _mTm(wk_l[c, g] * dw_all[g], vn_c[g]))

    S_ref[...] = jnp.concatenate(S_vals, axis=0)
    prev_ref[0:G, :] = jnp.concatenate(
        [rk_l[CPB - 1, g][C - 1:C, :] for g in range(G)], axis=0)


def kernel(out, W_write, W_read, W_beta, decay):
    B, T, D = out.shape
    H = decay.shape[0]
    d = D // H
    N = T // C
    BT = B * T
    f32 = jnp.float32

    x2 = out.reshape(BT, D).astype(f32)

    log_gamma = jnp.log(jax.nn.sigmoid(decay))
    pos = jnp.arange(C, dtype=f32)
    cum = (pos + 1.0) * log_gamma[:, None]
    tril = jnp.tril(jnp.ones((C, C), f32))
    L_mask = jnp.exp((cum[:, :, None] - cum[:, None, :]) * tril) * tril
    decay_exp = jnp.broadcast_to(jnp.exp(cum)[:, :, None], (H, C, d)) + 0.0
    chunk_total = jnp.exp(cum[:, -1])
    dw = jnp.broadcast_to(jnp.exp(cum[:, -1:] - cum)[:, :, None], (H, C, d)) + 0.0
    eh = jnp.repeat(jnp.eye(H, dtype=f32), d, axis=1)

    n_tiles = BT // TM
    v_flat = pl.pallas_call(
        _proj_in_body,
        grid=(n_tiles,),
        in_specs=[
            pl.BlockSpec((TM, D), lambda i: (i, 0)),
            pl.BlockSpec((D, D), lambda i: (0, 0)),
            pl.BlockSpec((H, D), lambda i: (0, 0)),
            pl.BlockSpec((H, D), lambda i: (0, 0)),
        ],
        out_specs=pl.BlockSpec((TM, D), lambda i: (i, 0)),
        out_shape=jax.ShapeDtypeStruct((BT, D), f32),
        compiler_params=pltpu.CompilerParams(
            dimension_semantics=("parallel",)),
        name="hebbian_proj_in",
    )(x2, W_write, W_beta, eh)

    BH = B * H
    scan_body = functools.partial(_scan_body, H=H, d=d)
    NB = N // CPB
    row_map = lambda p, n: (p // (H // G) * NB + n, jax.lax.rem(p, H // G))
    hd_map = lambda p, n: (jax.lax.rem(p, H // G), 0, 0)
    o_flat = pl.pallas_call(
        scan_body,
        grid=(BH // G, NB),
        in_specs=[
            pl.BlockSpec((CPB * C, G * d), row_map),
            pl.BlockSpec((CPB * C, G * d), row_map),
            pl.BlockSpec((H, D), lambda p, n: (0, 0)),
            pl.BlockSpec((H, D), lambda p, n: (0, 0)),
            pl.BlockSpec((G, C, C), hd_map),
            pl.BlockSpec((G, C, d), hd_map),
            pl.BlockSpec((G, C, d), hd_map),
            pl.BlockSpec(memory_space=pltpu.SMEM),
        ],
        out_specs=pl.BlockSpec((CPB * C, G * d), row_map),
        out_shape=jax.ShapeDtypeStruct((BT, D), f32),
        scratch_shapes=[
            pltpu.VMEM((G * d, d), f32),
            pltpu.VMEM((8, d), f32),
        ],
        compiler_params=pltpu.CompilerParams(
            dimension_semantics=("parallel", "arbitrary")),
        name="hebbian_scan",
    )(x2, v_flat, W_beta, eh, L_mask, decay_exp, dw, chunk_total)

    y = pl.pallas_call(
        _proj_out_body,
        grid=(n_tiles,),
        in_specs=[
            pl.BlockSpec((TM, D), lambda i: (i, 0)),
            pl.BlockSpec((TM, D), lambda i: (i, 0)),
            pl.BlockSpec((D, D), lambda i: (0, 0)),
        ],
        out_specs=pl.BlockSpec((TM, D), lambda i: (i, 0)),
        out_shape=jax.ShapeDtypeStruct((BT, D), f32),
        compiler_params=pltpu.CompilerParams(
            dimension_semantics=("parallel",)),
        name="hebbian_proj_out",
    )(o_flat, x2, W_read)

    return y.reshape(B, T, D).astype(out.dtype)


# speedup vs baseline: 1.1028x; 1.1028x over previous
import functools

import jax
import jax.numpy as jnp
from jax.experimental import pallas as pl
from jax.experimental.pallas import tpu as pltpu

C = 64
G = 8
CPB = 4
TM = 1024

_dot = functools.partial(jax.lax.dot_general,
                         preferred_element_type=jnp.float32,
                         precision=None)


def _mm(a, b):
    return _dot(a, b, (((1,), (0,)), ((), ())))


def _mmT(a, b):
    return _dot(a, b, (((1,), (1,)), ((), ())))


def _mTm(a, b):
    return _dot(a, b, (((0,), (0,)), ((), ())))


def _proj_in_body(x_ref, ww_ref, wb_ref, e_ref, v_ref, bexp_ref):
    x = x_ref[...]
    bexp = _mm(jax.nn.sigmoid(_mmT(x, wb_ref[...])), e_ref[...])
    bexp_ref[...] = bexp
    v_ref[...] = _mmT(x, ww_ref[...]) * bexp


def _proj_out_body(o_ref, out_ref, wr_ref, y_ref):
    y_ref[...] = out_ref[...] + _mmT(o_ref[...], wr_ref[...])


def _scan_body(x_ref, v_ref, bexp_ref, L_ref, dexp_ref, dw_ref, ct_ref,
               o_ref, S_ref, prev_ref, *, H, d):
    n = pl.program_id(1)

    @pl.when(n == 0)
    def _():
        S_ref[...] = jnp.zeros_like(S_ref)
        prev_ref[...] = jnp.zeros_like(prev_ref)

    ii = jax.lax.broadcasted_iota(jnp.int32, (C, C), 0)
    jj = jax.lax.broadcasted_iota(jnp.int32, (C, C), 1)
    strict = ii > jj

    hg = jax.lax.rem(pl.program_id(0), jnp.int32(H // G))

    S_all = S_ref[...]
    prev_all = prev_ref[...]
    x_all = x_ref[...]
    v_all = v_ref[...]
    b_all = bexp_ref[...]
    L_all = L_ref[...]
    de_all = dexp_ref[...]
    dw_all = dw_ref[...]

    rk_l, wk_l, Mk_l, X_l, attn_l = {}, {}, {}, {}, {}
    for c in range(CPB):
        rs = slice(c * C, (c + 1) * C)
        for g in range(G):
            sl = slice(g * d, (g + 1) * d)
            xh = x_all[rs, sl]
            n2 = jnp.sum(xh * xh, axis=1, keepdims=True)
            rk_l[c, g] = xh * jax.lax.rsqrt(jnp.maximum(n2, 1e-24))
    for c in range(CPB):
        for g in range(G):
            prev = (prev_all[g:g + 1, :] if c == 0
                    else rk_l[c - 1, g][C - 1:C, :])
            wk_l[c, g] = jnp.concatenate([prev, rk_l[c, g][:C - 1, :]], axis=0)
    for c in range(CPB):
        rs = slice(c * C, (c + 1) * C)
        for g in range(G):
            sl = slice(g * d, (g + 1) * d)
            wkb = wk_l[c, g] * b_all[rs, sl]
            pre = _mmT(jnp.concatenate([wkb, rk_l[c, g]], axis=0), wk_l[c, g])
            Mk_l[c, g] = jnp.where(strict, -pre[:C] * L_all[g], 0.0)
            attn_l[c, g] = pre[C:] * L_all[g]
            X_l[c, g] = jnp.concatenate([v_all[rs, sl], wkb * de_all[g]],
                                        axis=1)
    for c in range(CPB):
        for g in range(G):
            X_l[c, g] = X_l[c, g] + _mm(Mk_l[c, g], X_l[c, g])
    for _ in range(3):
        for c in range(CPB):
            for g in range(G):
                Mk_l[c, g] = _mm(Mk_l[c, g], Mk_l[c, g])
        for c in range(CPB):
            for g in range(G):
                X_l[c, g] = X_l[c, g] + _mm(Mk_l[c, g], X_l[c, g])
    M16_l = {}
    for c in range(CPB):
        for g in range(G):
            M16_l[c, g] = _mm(Mk_l[c, g][16:], Mk_l[c, g])
    for c in range(CPB):
        for g in range(G):
            X = X_l[c, g]
            X_l[c, g] = jnp.concatenate(
                [X[:16], X[16:] + _mm(M16_l[c, g], X)], axis=0)
    M32_l = {}
    for c in range(CPB):
        for g in range(G):
            M32_l[c, g] = _mm(M16_l[c, g][16:, 16:48],
                              M16_l[c, g][:32])
    for c in range(CPB):
        for g in range(G):
            X = X_l[c, g]
            X_l[c, g] = jnp.concatenate(
                [X[:32], X[32:] + _mm(M32_l[c, g][:, :32], X[:32])], axis=0)

    S_vals = [S_all[g * d:(g + 1) * d, :] for g in range(G)]
    for c in range(CPB):
        rs = slice(c * C, (c + 1) * C)
        vn_c = []
        for g in range(G):
            vn_c.append(X_l[c, g][:, :d] - _mm(X_l[c, g][:, d:], S_vals[g]))
        for g in range(G):
            sl = slice(g * d, (g + 1) * d)
            o_ref[rs, sl] = _mm(
                jnp.concatenate([rk_l[c, g] * de_all[g], attn_l[c, g]], axis=1),
                jnp.concatenate([S_vals[g], vn_c[g]], axis=0))
        for g in range(G):
            S_vals[g] = (ct_ref[hg * G + g] * S_vals[g]
                         + _mTm(wk_l[c, g] * dw_all[g], vn_c[g]))

    S_ref[...] = jnp.concatenate(S_vals, axis=0)
    prev_ref[0:G, :] = jnp.concatenate(
        [rk_l[CPB - 1, g][C - 1:C, :] for g in range(G)], axis=0)


def kernel(out, W_write, W_read, W_beta, decay):
    B, T, D = out.shape
    H = decay.shape[0]
    d = D // H
    N = T // C
    BT = B * T
    f32 = jnp.float32

    x2 = out.reshape(BT, D).astype(f32)

    log_gamma = jnp.log(jax.nn.sigmoid(decay))
    pos = jnp.arange(C, dtype=f32)
    cum = (pos + 1.0) * log_gamma[:, None]
    tril = jnp.tril(jnp.ones((C, C), f32))
    L_mask = jnp.exp((cum[:, :, None] - cum[:, None, :]) * tril) * tril
    decay_exp = jnp.broadcast_to(jnp.exp(cum)[:, :, None], (H, C, d)) + 0.0
    chunk_total = jnp.exp(cum[:, -1])
    dw = jnp.broadcast_to(jnp.exp(cum[:, -1:] - cum)[:, :, None], (H, C, d)) + 0.0
    eh = jnp.repeat(jnp.eye(H, dtype=f32), d, axis=1)

    n_tiles = BT // TM
    v_flat, bexp_flat = pl.pallas_call(
        _proj_in_body,
        grid=(n_tiles,),
        in_specs=[
            pl.BlockSpec((TM, D), lambda i: (i, 0)),
            pl.BlockSpec((D, D), lambda i: (0, 0)),
            pl.BlockSpec((H, D), lambda i: (0, 0)),
            pl.BlockSpec((H, D), lambda i: (0, 0)),
        ],
        out_specs=[
            pl.BlockSpec((TM, D), lambda i: (i, 0)),
            pl.BlockSpec((TM, D), lambda i: (i, 0)),
        ],
        out_shape=[
            jax.ShapeDtypeStruct((BT, D), f32),
            jax.ShapeDtypeStruct((BT, D), f32),
        ],
        compiler_params=pltpu.CompilerParams(
            dimension_semantics=("parallel",)),
        name="hebbian_proj_in",
    )(x2, W_write, W_beta, eh)

    BH = B * H
    scan_body = functools.partial(_scan_body, H=H, d=d)
    NB = N // CPB
    row_map = lambda p, n: (p // (H // G) * NB + n, jax.lax.rem(p, H // G))
    hd_map = lambda p, n: (jax.lax.rem(p, H // G), 0, 0)
    o_flat = pl.pallas_call(
        scan_body,
        grid=(BH // G, NB),
        in_specs=[
            pl.BlockSpec((CPB * C, G * d), row_map),
            pl.BlockSpec((CPB * C, G * d), row_map),
            pl.BlockSpec((CPB * C, G * d), row_map),
            pl.BlockSpec((G, C, C), hd_map),
            pl.BlockSpec((G, C, d), hd_map),
            pl.BlockSpec((G, C, d), hd_map),
            pl.BlockSpec(memory_space=pltpu.SMEM),
        ],
        out_specs=pl.BlockSpec((CPB * C, G * d), row_map),
        out_shape=jax.ShapeDtypeStruct((BT, D), f32),
        scratch_shapes=[
            pltpu.VMEM((G * d, d), f32),
            pltpu.VMEM((8, d), f32),
        ],
        compiler_params=pltpu.CompilerParams(
            dimension_semantics=("parallel", "arbitrary")),
        name="hebbian_scan",
    )(x2, v_flat, bexp_flat, L_mask, decay_exp, dw, chunk_total)

    y = pl.pallas_call(
        _proj_out_body,
        grid=(n_tiles,),
        in_specs=[
            pl.BlockSpec((TM, D), lambda i: (i, 0)),
            pl.BlockSpec((TM, D), lambda i: (i, 0)),
            pl.BlockSpec((D, D), lambda i: (0, 0)),
        ],
        out_specs=pl.BlockSpec((TM, D), lambda i: (i, 0)),
        out_shape=jax.ShapeDtypeStruct((BT, D), f32),
        compiler_params=pltpu.CompilerParams(
            dimension_semantics=("parallel",)),
        name="hebbian_proj_out",
    )(o_flat, x2, W_read)

    return y.reshape(B, T, D).astype(out.dtype)
